# baseline (speedup 1.0000x reference)
import functools

import jax
import jax.numpy as jnp
from jax import lax
from jax.experimental import pallas as pl
from jax.experimental.pallas import tpu as pltpu
from jax.experimental.pallas import tpu_sc as plsc

N = 50000
E = 800000
D = 64
NC = 2
NS = 16
L = 16

HALF = N // NC
TROWS = 25088
RPT = TROWS // NS
LASTR = HALF - (NS - 1) * RPT
TRASH = HALF

B = 128
EPT = 52224
NCH = EPT // B
SLOTS = 3
UNROLL = 6
NIT = NCH // UNROLL
GL = 2
E2 = NS * EPT
ERPT = EPT // B

_mesh = plsc.VectorSubcoreMesh(core_axis_name="c", subcore_axis_name="s")


@functools.partial(
    pl.kernel,
    out_type=jax.ShapeDtypeStruct((N, D), jnp.float32),
    mesh=_mesh,
    compiler_params=pltpu.CompilerParams(use_tc_tiling_on_sc=False,
                                         needs_layout_passes=False),
    scratch_types=[
        pltpu.VMEM_SHARED((TROWS, D), jnp.float32),
        pltpu.VMEM((SLOTS, B), jnp.int32),
        pltpu.VMEM((SLOTS, B), jnp.int32),
        pltpu.VMEM((2, B), jnp.int32),
        pltpu.VMEM((SLOTS, B, D), jnp.bfloat16),
        pltpu.VMEM((2, B, D), jnp.float32),
        pltpu.SemaphoreType.DMA,
        pltpu.SemaphoreType.DMA,
        pltpu.SemaphoreType.DMA,
    ],
)
def _sc_agg(h_hbm, src_hbm, dst_hbm, zeros_hbm, out_hbm,
            table, srcs, dsts, dstl, rows_bf, rows_f, isem, gsem, ssem):
    c = lax.axis_index("c")
    s = lax.axis_index("s")
    base = c * HALF

    pltpu.sync_copy(zeros_hbm, table.at[pl.ds(s * RPT, RPT)])
    plsc.subcore_barrier()

    def fire_idx(ch, slot):
        r0 = s * ERPT + ch
        pltpu.async_copy(src_hbm.at[pl.ds(r0, 1)], srcs.at[pl.ds(slot, 1)], isem)
        pltpu.async_copy(dst_hbm.at[pl.ds(r0, 1)], dsts.at[pl.ds(slot, 1)], isem)

    def wait_idx():
        for _ in range(2):
            pltpu.make_async_copy(src_hbm.at[pl.ds(0, 1)],
                                  srcs.at[pl.ds(0, 1)], isem).wait()

    def remap(islot, dslot):
        for jj in range(B // L):
            d = dsts[islot, pl.ds(jj * L, L)]
            m = (d >= base) & (d < base + HALF)
            dstl[dslot, pl.ds(jj * L, L)] = jnp.where(m, d - base, TRASH)

    def widen(slot, fslot):
        def wbody(i, carry):
            r8 = i * 8
            for dr in range(8):
                r = r8 + dr
                for g in range(D // 32):
                    v = plsc.bitcast(rows_bf[slot, r, pl.ds(32 * g, 32)],
                                     jnp.int32)
                    lo = plsc.bitcast(v << 16, jnp.float32)
                    hi = plsc.bitcast(v & jnp.int32(-65536), jnp.float32)
                    rows_f[fslot, r, pl.ds(32 * g, L)] = lo
                    rows_f[fslot, r, pl.ds(32 * g + L, L)] = hi
            return carry

        lax.fori_loop(0, B // 8, wbody, 0)

    def fire_gather(slot):
        pltpu.async_copy(h_hbm.at[srcs.at[slot]], rows_bf.at[slot], gsem)

    def wait_gather():
        pltpu.make_async_copy(h_hbm.at[srcs.at[0]], rows_bf.at[0], gsem).wait()

    def fire_scatter(fslot):
        pltpu.make_async_copy(rows_f.at[fslot], table.at[dstl.at[fslot]],
                              ssem).start(add=True)

    def wait_scatter():
        pltpu.make_async_copy(rows_f.at[0], table.at[dstl.at[0]], ssem).wait()

    for k in range(GL):
        fire_idx(k, k)
    for k in range(GL):
        wait_idx()
        fire_gather(k)
    fire_idx(GL, GL)

    def outer(t, carry):
        for u in range(UNROLL):
            ch = t * UNROLL + u
            b = u % SLOTS
            p = u % 2

            @pl.when(ch >= 2)
            def _():
                wait_scatter()

            wait_idx()
            fire_gather((b + GL) % SLOTS)
            wait_gather()
            remap(b, p)
            chp = jnp.minimum(ch + GL + 1, NCH - 1)
            fire_idx(chp, (b + GL + 1) % SLOTS)
            widen(b, p)
            fire_scatter(p)
        return carry

    lax.fori_loop(0, NIT, outer, 0)

    wait_idx()
    for _ in range(GL):
        wait_gather()
    wait_scatter()
    wait_scatter()

    plsc.subcore_barrier()

    @pl.when(s < NS - 1)
    def _():
        pltpu.sync_copy(table.at[pl.ds(s * RPT, RPT)],
                        out_hbm.at[pl.ds(base + s * RPT, RPT)])

    @pl.when(s == NS - 1)
    def _():
        pltpu.sync_copy(table.at[pl.ds(s * RPT, LASTR)],
                        out_hbm.at[pl.ds(base + s * RPT, LASTR)])


BN = 1024


def _mlp_body(x_ref, agg_ref, w1_ref, b1_ref, w2_ref, b2_ref, out_ref):
    h = x_ref[...] + agg_ref[...]
    h = jnp.tanh(jnp.dot(h, w1_ref[...], preferred_element_type=jnp.float32)
                 + b1_ref[...])
    out_ref[...] = (jnp.dot(h, w2_ref[...], preferred_element_type=jnp.float32)
                    + b2_ref[...])


def _mlp(x, agg, w1, b1, w2, b2):
    full = lambda i: (0, 0)
    blk = lambda i: (i, 0)
    return pl.pallas_call(
        _mlp_body,
        grid=(pl.cdiv(N, BN),),
        in_specs=[
            pl.BlockSpec((BN, D), blk),
            pl.BlockSpec((BN, D), blk),
            pl.BlockSpec((D, D), full),
            pl.BlockSpec((1, D), full),
            pl.BlockSpec((D, D), full),
            pl.BlockSpec((1, D), full),
        ],
        out_specs=pl.BlockSpec((BN, D), blk),
        out_shape=jax.ShapeDtypeStruct((N, D), jnp.float32),
    )(x, agg, w1, b1, w2, b2)


def _to_bf16_interleaved(h):
    h4 = h.reshape(N, 2, 2, L)
    return h4.transpose(0, 1, 3, 2).reshape(N, D).astype(jnp.bfloat16)


def kernel(x, edge_index, W1_0, b1_0, W2_0, b2_0, W1_1, b1_1, W2_1, b2_1):
    src = edge_index[0].astype(jnp.int32)
    dst = edge_index[1].astype(jnp.int32)
    pad = E2 - E
    src = jnp.concatenate([src, jnp.zeros((pad,), jnp.int32)]).reshape(E2 // B, B)
    dst = jnp.concatenate([dst, jnp.full((pad,), N, jnp.int32)]).reshape(E2 // B, B)
    zeros = jnp.zeros((RPT, D), jnp.float32)
    h = x
    for (w1, b1, w2, b2) in ((W1_0, b1_0, W2_0, b2_0),
                             (W1_1, b1_1, W2_1, b2_1)):
        agg = _sc_agg(_to_bf16_interleaved(h), src, dst, zeros)
        h = _mlp(h, agg, w1, b1.reshape(1, D), w2, b2.reshape(1, D))
    return h


# speedup vs baseline: 1.1740x; 1.1740x over previous
import functools

import jax
import jax.numpy as jnp
from jax import lax
from jax.experimental import pallas as pl
from jax.experimental.pallas import tpu as pltpu
from jax.experimental.pallas import tpu_sc as plsc

N = 50000
E = 800000
D = 64
NC = 2
NS = 16
L = 16

HALF = N // NC
TROWS = 25088
RPT = TROWS // NS
LASTR = HALF - (NS - 1) * RPT
TRASH = HALF

B = 128
EPT = 51200
NCH = EPT // B
SLOTS = 2
UNROLL = 2
NIT = NCH // UNROLL
GL = 1
E2 = NS * EPT
ERPT = EPT // B

_mesh = plsc.VectorSubcoreMesh(core_axis_name="c", subcore_axis_name="s")


@functools.partial(
    pl.kernel,
    out_type=jax.ShapeDtypeStruct((N, D), jnp.float32),
    mesh=_mesh,
    compiler_params=pltpu.CompilerParams(use_tc_tiling_on_sc=False,
                                         needs_layout_passes=False),
    scratch_types=[
        pltpu.VMEM_SHARED((TROWS, D), jnp.float32),
        pltpu.VMEM((SLOTS, B), jnp.int32),
        pltpu.VMEM((SLOTS, B), jnp.int32),
        pltpu.VMEM((2, B), jnp.int32),
        pltpu.VMEM((SLOTS, B, D), jnp.bfloat16),
        pltpu.VMEM((2, B, D), jnp.float32),
        pltpu.SemaphoreType.DMA,
        pltpu.SemaphoreType.DMA,
        pltpu.SemaphoreType.DMA,
    ],
)
def _sc_agg(h_hbm, src_hbm, dst_hbm, zeros_hbm, out_hbm,
            table, srcs, dsts, dstl, rows_bf, rows_f, isem, gsem, ssem):
    c = lax.axis_index("c")
    s = lax.axis_index("s")
    base = c * HALF

    pltpu.sync_copy(zeros_hbm, table.at[pl.ds(s * RPT, RPT)])
    plsc.subcore_barrier()

    def fire_idx(ch, slot):
        r0 = s * ERPT + ch
        pltpu.async_copy(src_hbm.at[pl.ds(r0, 1)], srcs.at[pl.ds(slot, 1)], isem)
        pltpu.async_copy(dst_hbm.at[pl.ds(r0, 1)], dsts.at[pl.ds(slot, 1)], isem)

    def wait_idx():
        for _ in range(2):
            pltpu.make_async_copy(src_hbm.at[pl.ds(0, 1)],
                                  srcs.at[pl.ds(0, 1)], isem).wait()

    def remap(islot, dslot):
        for jj in range(B // L):
            d = dsts[islot, pl.ds(jj * L, L)]
            m = (d >= base) & (d < base + HALF)
            dstl[dslot, pl.ds(jj * L, L)] = jnp.where(m, d - base, TRASH)

    def widen(slot, fslot):
        for r in range(B):
            for g in range(D // 32):
                v = plsc.bitcast(rows_bf[slot, r, pl.ds(32 * g, 32)],
                                 jnp.int32)
                lo = plsc.bitcast(v << 16, jnp.float32)
                hi = plsc.bitcast(v & jnp.int32(-65536), jnp.float32)
                rows_f[fslot, r, pl.ds(32 * g, L)] = lo
                rows_f[fslot, r, pl.ds(32 * g + L, L)] = hi

    def fire_gather(slot):
        pltpu.async_copy(h_hbm.at[srcs.at[slot]], rows_bf.at[slot], gsem)

    def wait_gather():
        pltpu.make_async_copy(h_hbm.at[srcs.at[0]], rows_bf.at[0], gsem).wait()

    def fire_scatter(fslot):
        pltpu.make_async_copy(rows_f.at[fslot], table.at[dstl.at[fslot]],
                              ssem).start(add=True)

    def wait_scatter():
        pltpu.make_async_copy(rows_f.at[0], table.at[dstl.at[0]], ssem).wait()

    fire_idx(0, 0)
    fire_idx(1, 1)
    wait_idx()
    fire_gather(0)

    def outer(t, carry):
        for q in range(UNROLL):
            ch = t * UNROLL + q

            @pl.when(ch >= 2)
            def _():
                wait_scatter()

            wait_gather()
            wait_idx()
            fire_gather(1 - q)
            remap(q, q)
            chp = jnp.minimum(ch + 2, NCH - 1)
            fire_idx(chp, q)
            widen(q, q)
            fire_scatter(q)
        return carry

    lax.fori_loop(0, NIT, outer, 0)

    wait_idx()
    wait_gather()
    wait_scatter()
    wait_scatter()

    plsc.subcore_barrier()

    @pl.when(s < NS - 1)
    def _():
        pltpu.sync_copy(table.at[pl.ds(s * RPT, RPT)],
                        out_hbm.at[pl.ds(base + s * RPT, RPT)])

    @pl.when(s == NS - 1)
    def _():
        pltpu.sync_copy(table.at[pl.ds(s * RPT, LASTR)],
                        out_hbm.at[pl.ds(base + s * RPT, LASTR)])


BN = 1024


def _mlp_body(x_ref, agg_ref, w1_ref, b1_ref, w2_ref, b2_ref, out_ref):
    h = x_ref[...] + agg_ref[...]
    h = jnp.tanh(jnp.dot(h, w1_ref[...], preferred_element_type=jnp.float32)
                 + b1_ref[...])
    out_ref[...] = (jnp.dot(h, w2_ref[...], preferred_element_type=jnp.float32)
                    + b2_ref[...])


def _mlp(x, agg, w1, b1, w2, b2):
    full = lambda i: (0, 0)
    blk = lambda i: (i, 0)
    return pl.pallas_call(
        _mlp_body,
        grid=(pl.cdiv(N, BN),),
        in_specs=[
            pl.BlockSpec((BN, D), blk),
            pl.BlockSpec((BN, D), blk),
            pl.BlockSpec((D, D), full),
            pl.BlockSpec((1, D), full),
            pl.BlockSpec((D, D), full),
            pl.BlockSpec((1, D), full),
        ],
        out_specs=pl.BlockSpec((BN, D), blk),
        out_shape=jax.ShapeDtypeStruct((N, D), jnp.float32),
    )(x, agg, w1, b1, w2, b2)


def _to_bf16_interleaved(h):
    h4 = h.reshape(N, 2, 2, L)
    return h4.transpose(0, 1, 3, 2).reshape(N, D).astype(jnp.bfloat16)


def kernel(x, edge_index, W1_0, b1_0, W2_0, b2_0, W1_1, b1_1, W2_1, b2_1):
    src = edge_index[0].astype(jnp.int32)
    dst = edge_index[1].astype(jnp.int32)
    pad = E2 - E
    src = jnp.concatenate([src, jnp.zeros((pad,), jnp.int32)]).reshape(E2 // B, B)
    dst = jnp.concatenate([dst, jnp.full((pad,), N, jnp.int32)]).reshape(E2 // B, B)
    zeros = jnp.zeros((RPT, D), jnp.float32)
    h = x
    for (w1, b1, w2, b2) in ((W1_0, b1_0, W2_0, b2_0),
                             (W1_1, b1_1, W2_1, b2_1)):
        agg = _sc_agg(_to_bf16_interleaved(h), src, dst, zeros)
        h = _mlp(h, agg, w1, b1.reshape(1, D), w2, b2.reshape(1, D))
    return h
